# baseline (speedup 1.0000x reference)
import functools

import jax
import jax.numpy as jnp
from jax import lax
from jax.experimental import pallas as pl
from jax.experimental.pallas import tpu as pltpu
from jax.experimental.pallas import tpu_sc as plsc

N_ATOMS = 4096
BOX = 40.0
CUT = 5.0
HIDDEN = 64
NG = 50
N_CONV = 3
NW = 32
APW = N_ATOMS // NW
CAP = 8192
EC = 128
TRASH = APW
ACC_ROWS = APW + 8
BS = 2048

_f32 = jnp.float32
_i32 = jnp.int32


def _ssp(x):
    return jnp.maximum(x, 0.0) + jnp.log1p(jnp.exp(-jnp.abs(x))) - 0.6931471805599453


def _full(v, dtype=_i32):
    return jnp.full((16,), v, dtype=dtype)


CAPA = 96


def _nbr_call(xs, ys, zs):
    mesh = plsc.VectorSubcoreMesh(core_axis_name="c", subcore_axis_name="s")

    @functools.partial(
        pl.kernel,
        mesh=mesh,
        compiler_params=pltpu.CompilerParams(needs_layout_passes=False),
        out_type=[
            jax.ShapeDtypeStruct((NW * CAP,), _i32),
            jax.ShapeDtypeStruct((NW * CAP,), _i32),
            jax.ShapeDtypeStruct((NW * CAP,), _f32),
            jax.ShapeDtypeStruct((NW * 16,), _i32),
        ],
        scratch_types=[
            pltpu.VMEM((N_ATOMS,), _f32),
            pltpu.VMEM((N_ATOMS,), _f32),
            pltpu.VMEM((N_ATOMS,), _f32),
            pltpu.VMEM((APW * CAPA,), _i32),
            pltpu.VMEM((APW * CAPA,), _f32),
            pltpu.VMEM((APW,), _i32),
            pltpu.VMEM((CAP,), _i32),
            pltpu.VMEM((CAP,), _i32),
            pltpu.VMEM((CAP,), _f32),
            pltpu.VMEM((16,), _i32),
        ],
    )
    def nbr(xs_hbm, ys_hbm, zs_hbm, src_hbm, dst_hbm, d2_hbm, cnt_hbm,
            x_v, y_v, z_v, jb_v, d2b_v, ca_v, src_v, dst_v, d2_v, cnt_v):
        wid = lax.axis_index("s") * 2 + lax.axis_index("c")
        pltpu.sync_copy(xs_hbm, x_v)
        pltpu.sync_copy(ys_hbm, y_v)
        pltpu.sync_copy(zs_hbm, z_v)

        lane = lax.iota(_i32, 16)

        def group_body(g, _):
            base = wid * APW + g * 16
            dst_ids = _full(base) + lane
            xd = x_v[pl.ds(base, 16)]
            yd = y_v[pl.ds(base, 16)]
            zd = z_v[pl.ds(base, 16)]
            slot_base = (_full(g * 16) + lane) * CAPA

            def blk_body(blk, cnt_vec):
                j0 = blk * 16
                jvs, d2s, msks = [], [], []
                for j2 in range(16):
                    jv = _full(j2) + j0
                    dx = jnp.abs(xd - plsc.load_gather(x_v, [jv]))
                    dy = jnp.abs(yd - plsc.load_gather(y_v, [jv]))
                    dz = jnp.abs(zd - plsc.load_gather(z_v, [jv]))
                    dx = jnp.minimum(dx, BOX - dx)
                    dy = jnp.minimum(dy, BOX - dy)
                    dz = jnp.minimum(dz, BOX - dz)
                    d2 = dx * dx + dy * dy + dz * dz
                    jvs.append(jv)
                    d2s.append(d2)
                    msks.append((d2 < CUT * CUT) & (dst_ids != jv))
                for j2 in range(16):
                    slot = slot_base + jnp.minimum(cnt_vec, CAPA - 1)
                    plsc.store_scatter(jb_v, [slot], jvs[j2], mask=msks[j2])
                    plsc.store_scatter(d2b_v, [slot], d2s[j2], mask=msks[j2])
                    cnt_vec = cnt_vec + msks[j2].astype(_i32)
                return cnt_vec

            cnt_vec = lax.fori_loop(0, N_ATOMS // 16, blk_body,
                                    jnp.zeros((16,), _i32))
            ca_v[pl.ds(g * 16, 16)] = jnp.minimum(cnt_vec, CAPA)
            return 0

        lax.fori_loop(0, APW // 16, group_body, 0)

        def compact_body(a_loc, cnt):
            n = jnp.max(plsc.load_gather(ca_v, [_full(a_loc)]))

            def copy_body(t, _):
                o = t * 16
                src_v[pl.ds(cnt + o, 16)] = jb_v[pl.ds(a_loc * CAPA + o, 16)]
                d2_v[pl.ds(cnt + o, 16)] = d2b_v[pl.ds(a_loc * CAPA + o, 16)]
                dst_v[pl.ds(cnt + o, 16)] = _full(0) + a_loc
                return 0

            lax.fori_loop(0, (n + 15) // 16, copy_body, 0)
            return jnp.minimum(cnt + n, CAP - 128)

        cnt = lax.fori_loop(0, APW, compact_body, jnp.int32(0))

        def sent_body(t, _):
            o = cnt + t * 16
            src_v[pl.ds(o, 16)] = _full(0)
            dst_v[pl.ds(o, 16)] = _full(TRASH)
            d2_v[pl.ds(o, 16)] = _full(1.0e9, _f32)
            return 0

        lax.fori_loop(0, EC // 16, sent_body, 0)

        pltpu.sync_copy(src_v, src_hbm.at[pl.ds(wid * CAP, CAP)])
        pltpu.sync_copy(dst_v, dst_hbm.at[pl.ds(wid * CAP, CAP)])
        pltpu.sync_copy(d2_v, d2_hbm.at[pl.ds(wid * CAP, CAP)])
        cnt_v[...] = jnp.full((16,), cnt, _i32)
        pltpu.sync_copy(cnt_v, cnt_hbm.at[pl.ds(wid * 16, 16)])

    return nbr(xs, ys, zs)


def _filters_kernel(cnts_ref, d2_ref, Wf1T_ref, bf1_ref, Wf2T_ref, bf2_ref,
                    f0_ref, f1_ref, f2_ref):
    b = pl.program_id(1)
    cnt = cnts_ref[pl.program_id(0) * 16]
    nb = jnp.maximum((cnt + BS - 1) // BS, 1)

    @pl.when(b < nb)
    def _():
        d2 = d2_ref[...].reshape(1, BS)
        d = jnp.sqrt(d2 + 1e-12)
        db = jnp.broadcast_to(d, (NG, BS))
        centers = lax.broadcasted_iota(_i32, (NG, BS), 0).astype(_f32) * (CUT / (NG - 1))
        delta = db - centers
        ef = jnp.exp(delta * delta * (-1.0 / (2.0 * (CUT / NG) ** 2)))
        for l, f_ref in enumerate((f0_ref, f1_ref, f2_ref)):
            u = _ssp(lax.dot_general(Wf1T_ref[l], ef, (((1,), (0,)), ((), ())),
                                     preferred_element_type=_f32, precision=lax.Precision.HIGHEST) + bf1_ref[l])
            fT = lax.dot_general(Wf2T_ref[l], u, (((1,), (0,)), ((), ())),
                                 preferred_element_type=_f32, precision=lax.Precision.HIGHEST) + bf2_ref[l]
            f_ref[...] = fT.reshape(1, HIDDEN, BS)


def _filters_call(cnts, d2, Wf1T, bf1c, Wf2T, bf2c):
    def clamp(w, b, cnts_sref):
        cnt = cnts_sref[w * 16]
        nb = jnp.maximum((cnt + BS - 1) // BS, 1)
        return jnp.minimum(b, nb - 1)

    grid_spec = pltpu.PrefetchScalarGridSpec(
        num_scalar_prefetch=1,
        grid=(NW, CAP // BS),
        in_specs=[
            pl.BlockSpec((1, 1, BS),
                         lambda w, b, c: (w * (CAP // BS) + clamp(w, b, c), 0, 0)),
            pl.BlockSpec((N_CONV, HIDDEN, NG), lambda w, b, c: (0, 0, 0)),
            pl.BlockSpec((N_CONV, HIDDEN, 1), lambda w, b, c: (0, 0, 0)),
            pl.BlockSpec((N_CONV, HIDDEN, HIDDEN), lambda w, b, c: (0, 0, 0)),
            pl.BlockSpec((N_CONV, HIDDEN, 1), lambda w, b, c: (0, 0, 0)),
        ],
        out_specs=[
            pl.BlockSpec((1, HIDDEN, BS),
                         lambda w, b, c: (w, 0, clamp(w, b, c)))
            for _ in range(N_CONV)
        ],
    )
    return pl.pallas_call(
        _filters_kernel,
        grid_spec=grid_spec,
        out_shape=[jax.ShapeDtypeStruct((NW, HIDDEN, CAP), _f32)
                   for _ in range(N_CONV)],
    )(cnts, d2.reshape(NW * (CAP // BS), 1, BS), Wf1T, bf1c, Wf2T, bf2c)


def _h_kernel(first, z_ref, emb_ref, hprev_ref, msg_ref, Wc2_ref, bc2_ref,
              Wc1_ref, bc1_ref, h_ref, hj_ref):
    if first:
        zcol = z_ref[...]
        oh = (zcol == lax.broadcasted_iota(_i32, (N_ATOMS, 10), 1)).astype(_f32)
        h = lax.dot_general(oh, emb_ref[...], (((1,), (0,)), ((), ())),
                            preferred_element_type=_f32, precision=lax.Precision.HIGHEST)
    else:
        m = lax.dot_general(msg_ref[...], Wc2_ref[...], (((0,), (0,)), ((), ())),
                            preferred_element_type=_f32, precision=lax.Precision.HIGHEST)
        h = hprev_ref[...] + _ssp(m + bc2_ref[...])
    h_ref[...] = h
    hj_ref[...] = lax.dot_general(
        Wc1_ref[...], h, (((0,), (1,)), ((), ())),
        preferred_element_type=_f32, precision=lax.Precision.HIGHEST) + bc1_ref[...]


def _h_call(first, z2, emb, hprev, msg, Wc2, bc2, Wc1, bc1r):
    return pl.pallas_call(
        functools.partial(_h_kernel, first),
        out_shape=[
            jax.ShapeDtypeStruct((N_ATOMS, HIDDEN), _f32),
            jax.ShapeDtypeStruct((HIDDEN, N_ATOMS), _f32),
        ],
    )(z2, emb, hprev, msg, Wc2, bc2, Wc1, bc1r)


def _msg_call(src, dst, cnts, f, hjT):
    mesh = plsc.VectorSubcoreMesh(core_axis_name="c", subcore_axis_name="s")
    NSEG = 4
    AROW = 256

    @functools.partial(
        pl.kernel,
        mesh=mesh,
        compiler_params=pltpu.CompilerParams(needs_layout_passes=False),
        out_type=jax.ShapeDtypeStruct((HIDDEN, N_ATOMS), _f32),
        scratch_types=[
            pltpu.VMEM((16, N_ATOMS), _f32),
            pltpu.VMEM((2 * EC,), _i32),
            pltpu.VMEM((2 * EC,), _i32),
            pltpu.VMEM((2 * 16, EC), _f32),
            pltpu.VMEM((16, NSEG * AROW), _f32),
            pltpu.VMEM((16,), _i32),
            pltpu.SemaphoreType.DMA,
            pltpu.SemaphoreType.DMA,
            pltpu.SemaphoreType.DMA,
        ],
    )
    def msg_k(src_hbm, dst_hbm, cnt_hbm, f_hbm, hjT_hbm, msgT_hbm,
              hjq_v, idx_v, dst_v, f_v, acc_v, cnt_v, semH, semL0, semL1):
        wid = lax.axis_index("s") * 2 + lax.axis_index("c")
        g = wid // NSEG
        q = wid % NSEG
        semL = [semL0, semL1]
        ARTOT = NSEG * AROW

        stage = pltpu.make_async_copy(
            hjT_hbm.at[pl.ds(q * 16, 16), :], hjq_v, semH)
        stage.start()

        for r in range(16):
            def zero_body(k, _, r=r):
                acc_v[r, pl.ds(k * 16, 16)] = jnp.zeros((16,), _f32)
                return 0

            lax.fori_loop(0, ARTOT // 16, zero_body, 0, unroll=4)
        stage.wait()

        lane = lax.iota(_i32, 16)
        f_row = [_full(sl * 16) + lane for sl in range(2)]

        for ksub in range(NSEG):
            seg = g * NSEG + ksub
            pltpu.sync_copy(cnt_hbm.at[pl.ds(seg * 16, 16)], cnt_v)
            cnt = jnp.max(cnt_v[...])
            nchunks = (cnt + (EC - 1)) // EC
            acc_col0 = ksub * AROW

            def L_descr(c, sl):
                e0 = c * EC
                return (
                    pltpu.make_async_copy(
                        src_hbm.at[pl.ds(seg * CAP + e0, EC)],
                        idx_v.at[pl.ds(sl * EC, EC)], semL[sl]),
                    pltpu.make_async_copy(
                        dst_hbm.at[pl.ds(seg * CAP + e0, EC)],
                        dst_v.at[pl.ds(sl * EC, EC)], semL[sl]),
                    pltpu.make_async_copy(
                        f_hbm.at[seg, pl.ds(q * 16, 16), pl.ds(e0, EC)],
                        f_v.at[pl.ds(sl * 16, 16), :], semL[sl]),
                )

            def startL(c, sl):
                for dsc in L_descr(c, sl):
                    dsc.start()

            def waitL(c, sl):
                for dsc in L_descr(c, sl):
                    dsc.wait()

            def compute(c, sl):
                def group_body(gi, _):
                    for k in range(16):
                        e = gi * 16 + k
                        ev = _full(sl * EC) + e
                        ecol = _full(0) + e
                        srcs = plsc.load_gather(idx_v, [ev])
                        dsts = plsc.load_gather(dst_v, [ev])
                        hjv = plsc.load_gather(hjq_v, [lane, srcs])
                        fv = plsc.load_gather(f_v, [f_row[sl], ecol])
                        plsc.addupdate_scatter(
                            acc_v, [lane, _full(acc_col0) + dsts], fv * hjv)
                    return 0

                lax.fori_loop(0, EC // 16, group_body, 0)

            @pl.when(nchunks > 0)
            def _():
                startL(0, 0)

            def super_body(qi, _):
                c0 = qi * 2
                for j in range(2):
                    c = c0 + j

                    @pl.when(c + 1 < nchunks)
                    def _():
                        startL(c + 1, (j + 1) % 2)

                    @pl.when(c < nchunks)
                    def _():
                        waitL(c, j)
                        compute(c, j)
                return 0

            lax.fori_loop(0, (nchunks + 1) // 2, super_body, 0)

        for ksub in range(NSEG):
            pltpu.sync_copy(
                acc_v.at[:, pl.ds(ksub * AROW, APW)],
                msgT_hbm.at[pl.ds(q * 16, 16),
                            pl.ds((g * NSEG + ksub) * APW, APW)])

    return msg_k(src, dst, cnts, f, hjT)


def _final_kernel(h_ref, msg_ref, Wc2_ref, bc2_ref, Wo1_ref, bo1_ref,
                  Wo2_ref, bo2_ref, out_ref):
    m = lax.dot_general(msg_ref[...], Wc2_ref[...], (((0,), (0,)), ((), ())),
                        preferred_element_type=_f32, precision=lax.Precision.HIGHEST)
    h = h_ref[...] + _ssp(m + bc2_ref[...])
    a1 = _ssp(lax.dot_general(h, Wo1_ref[...], (((1,), (0,)), ((), ())),
                              preferred_element_type=_f32, precision=lax.Precision.HIGHEST) + bo1_ref[...])
    e = lax.dot_general(a1, Wo2_ref[...], (((1,), (0,)), ((), ())),
                        preferred_element_type=_f32, precision=lax.Precision.HIGHEST) + bo2_ref[...]
    out_ref[...] = jnp.sum(e).reshape(1, 1)


def _final_call(h, msg, Wc2, bc2, Wo1, bo1r, Wo2, bo2r):
    return pl.pallas_call(
        _final_kernel,
        out_shape=jax.ShapeDtypeStruct((1, 1), _f32),
    )(h, msg, Wc2, bc2, Wo1, bo1r, Wo2, bo2r)


def kernel(xyz, emb, Wf1, bf1, Wf2, bf2, Wc1, bc1, Wc2, bc2, Wo1, bo1, Wo2, bo2, z):
    xyzf = xyz.astype(_f32)
    src_a, dst_a, d2, cnts = _nbr_call(xyzf[:, 0], xyzf[:, 1], xyzf[:, 2])

    z2 = z.astype(_i32).reshape(N_ATOMS, 1)
    dummy_h = jnp.zeros((N_ATOMS, HIDDEN), _f32)
    dummy_w = jnp.zeros((HIDDEN, HIDDEN), _f32)
    dummy_b = jnp.zeros((1, HIDDEN), _f32)

    Wf1T = jnp.transpose(Wf1, (0, 2, 1))
    bf1c = bf1.reshape(N_CONV, HIDDEN, 1)
    Wf2T = jnp.transpose(Wf2, (0, 2, 1))
    bf2c = bf2.reshape(N_CONV, HIDDEN, 1)

    fs = _filters_call(cnts, d2, Wf1T, bf1c, Wf2T, bf2c)

    h = dummy_h
    msg = jnp.zeros((HIDDEN, N_ATOMS), _f32)
    for l in range(N_CONV):
        first = l == 0
        h, hjT = _h_call(
            first, z2, emb, h, msg,
            dummy_w if first else Wc2[l - 1],
            dummy_b if first else bc2[l - 1].reshape(1, HIDDEN),
            Wc1[l], bc1[l].reshape(HIDDEN, 1),
        )
        msg = _msg_call(src_a, dst_a, cnts, fs[l], hjT)

    out = _final_call(h, msg, Wc2[N_CONV - 1], bc2[N_CONV - 1].reshape(1, HIDDEN),
                      Wo1, bo1.reshape(1, HIDDEN // 2), Wo2, bo2.reshape(1, 1))
    return out[0, 0]


# speedup vs baseline: 1.7706x; 1.7706x over previous
import functools

import jax
import jax.numpy as jnp
from jax import lax
from jax.experimental import pallas as pl
from jax.experimental.pallas import tpu as pltpu
from jax.experimental.pallas import tpu_sc as plsc

N_ATOMS = 4096
BOX = 40.0
CUT = 5.0
HIDDEN = 64
NG = 50
N_CONV = 3
NW = 32
APW = N_ATOMS // NW
CAP = 8192
EC = 128
TRASH = APW
ACC_ROWS = APW + 8
BS = 2048

_f32 = jnp.float32
_i32 = jnp.int32


def _ssp(x):
    return jnp.maximum(x, 0.0) + jnp.log1p(jnp.exp(-jnp.abs(x))) - 0.6931471805599453


def _full(v, dtype=_i32):
    return jnp.full((16,), v, dtype=dtype)


CAPA = 96


def _nbr_call(xs, ys, zs):
    mesh = plsc.VectorSubcoreMesh(core_axis_name="c", subcore_axis_name="s")

    @functools.partial(
        pl.kernel,
        mesh=mesh,
        compiler_params=pltpu.CompilerParams(needs_layout_passes=False),
        out_type=[
            jax.ShapeDtypeStruct((NW * CAP,), _i32),
            jax.ShapeDtypeStruct((NW * CAP,), _i32),
            jax.ShapeDtypeStruct((NW * CAP,), _f32),
            jax.ShapeDtypeStruct((NW * 16,), _i32),
        ],
        scratch_types=[
            pltpu.VMEM((N_ATOMS,), _f32),
            pltpu.VMEM((N_ATOMS,), _f32),
            pltpu.VMEM((N_ATOMS,), _f32),
            pltpu.VMEM((APW * CAPA,), _i32),
            pltpu.VMEM((APW * CAPA,), _f32),
            pltpu.VMEM((APW,), _i32),
            pltpu.VMEM((CAP,), _i32),
            pltpu.VMEM((CAP,), _i32),
            pltpu.VMEM((CAP,), _f32),
            pltpu.VMEM((16,), _i32),
        ],
    )
    def nbr(xs_hbm, ys_hbm, zs_hbm, src_hbm, dst_hbm, d2_hbm, cnt_hbm,
            x_v, y_v, z_v, jb_v, d2b_v, ca_v, src_v, dst_v, d2_v, cnt_v):
        wid = lax.axis_index("s") * 2 + lax.axis_index("c")
        pltpu.sync_copy(xs_hbm, x_v)
        pltpu.sync_copy(ys_hbm, y_v)
        pltpu.sync_copy(zs_hbm, z_v)

        lane = lax.iota(_i32, 16)

        def group_body(g, _):
            base = wid * APW + g * 16
            dst_ids = _full(base) + lane
            xd = x_v[pl.ds(base, 16)]
            yd = y_v[pl.ds(base, 16)]
            zd = z_v[pl.ds(base, 16)]
            slot_base = (_full(g * 16) + lane) * CAPA

            def blk_body(blk, cnt_vec):
                j0 = blk * 16
                jvs, d2s, msks = [], [], []
                for j2 in range(16):
                    jv = _full(j2) + j0
                    dx = jnp.abs(xd - plsc.load_gather(x_v, [jv]))
                    dy = jnp.abs(yd - plsc.load_gather(y_v, [jv]))
                    dz = jnp.abs(zd - plsc.load_gather(z_v, [jv]))
                    dx = jnp.minimum(dx, BOX - dx)
                    dy = jnp.minimum(dy, BOX - dy)
                    dz = jnp.minimum(dz, BOX - dz)
                    d2 = dx * dx + dy * dy + dz * dz
                    jvs.append(jv)
                    d2s.append(d2)
                    msks.append((d2 < CUT * CUT) & (dst_ids != jv))
                for j2 in range(16):
                    slot = slot_base + jnp.minimum(cnt_vec, CAPA - 1)
                    plsc.store_scatter(jb_v, [slot], jvs[j2], mask=msks[j2])
                    plsc.store_scatter(d2b_v, [slot], d2s[j2], mask=msks[j2])
                    cnt_vec = cnt_vec + msks[j2].astype(_i32)
                return cnt_vec

            cnt_vec = lax.fori_loop(0, N_ATOMS // 16, blk_body,
                                    jnp.zeros((16,), _i32))
            ca_v[pl.ds(g * 16, 16)] = jnp.minimum(cnt_vec, CAPA)
            return 0

        lax.fori_loop(0, APW // 16, group_body, 0)

        def compact_body(a_loc, cnt):
            n = jnp.max(plsc.load_gather(ca_v, [_full(a_loc)]))

            def copy_body(t, _):
                o = t * 16
                src_v[pl.ds(cnt + o, 16)] = jb_v[pl.ds(a_loc * CAPA + o, 16)]
                d2_v[pl.ds(cnt + o, 16)] = d2b_v[pl.ds(a_loc * CAPA + o, 16)]
                dst_v[pl.ds(cnt + o, 16)] = _full(0) + a_loc
                return 0

            lax.fori_loop(0, (n + 15) // 16, copy_body, 0)
            return jnp.minimum(cnt + n, CAP - 128)

        cnt = lax.fori_loop(0, APW, compact_body, jnp.int32(0))

        def sent_body(t, _):
            o = cnt + t * 16
            src_v[pl.ds(o, 16)] = _full(0)
            dst_v[pl.ds(o, 16)] = _full(TRASH)
            d2_v[pl.ds(o, 16)] = _full(1.0e9, _f32)
            return 0

        lax.fori_loop(0, EC // 16, sent_body, 0)

        pltpu.sync_copy(src_v, src_hbm.at[pl.ds(wid * CAP, CAP)])
        pltpu.sync_copy(dst_v, dst_hbm.at[pl.ds(wid * CAP, CAP)])
        pltpu.sync_copy(d2_v, d2_hbm.at[pl.ds(wid * CAP, CAP)])
        cnt_v[...] = jnp.full((16,), cnt, _i32)
        pltpu.sync_copy(cnt_v, cnt_hbm.at[pl.ds(wid * 16, 16)])

    return nbr(xs, ys, zs)


def _filters_kernel(cnts_ref, d2_ref, Wf1T_ref, bf1_ref, Wf2T_ref, bf2_ref,
                    f0_ref, f1_ref, f2_ref):
    b = pl.program_id(1)
    cnt = cnts_ref[pl.program_id(0) * 16]
    nb = jnp.maximum((cnt + BS - 1) // BS, 1)

    @pl.when(b < nb)
    def _():
        d2 = d2_ref[...].reshape(1, BS)
        d = jnp.sqrt(d2 + 1e-12)
        db = jnp.broadcast_to(d, (NG, BS))
        centers = lax.broadcasted_iota(_i32, (NG, BS), 0).astype(_f32) * (CUT / (NG - 1))
        delta = db - centers
        ef = jnp.exp(delta * delta * (-1.0 / (2.0 * (CUT / NG) ** 2)))
        for l, f_ref in enumerate((f0_ref, f1_ref, f2_ref)):
            u = _ssp(lax.dot_general(Wf1T_ref[l], ef, (((1,), (0,)), ((), ())),
                                     preferred_element_type=_f32, precision=lax.Precision.HIGHEST) + bf1_ref[l])
            fT = lax.dot_general(Wf2T_ref[l], u, (((1,), (0,)), ((), ())),
                                 preferred_element_type=_f32, precision=lax.Precision.HIGHEST) + bf2_ref[l]
            f_ref[...] = fT.reshape(1, HIDDEN, BS)


def _filters_call(cnts, d2, Wf1T, bf1c, Wf2T, bf2c):
    def clamp(w, b, cnts_sref):
        cnt = cnts_sref[w * 16]
        nb = jnp.maximum((cnt + BS - 1) // BS, 1)
        return jnp.minimum(b, nb - 1)

    grid_spec = pltpu.PrefetchScalarGridSpec(
        num_scalar_prefetch=1,
        grid=(NW, CAP // BS),
        in_specs=[
            pl.BlockSpec((1, 1, BS),
                         lambda w, b, c: (w * (CAP // BS) + clamp(w, b, c), 0, 0)),
            pl.BlockSpec((N_CONV, HIDDEN, NG), lambda w, b, c: (0, 0, 0)),
            pl.BlockSpec((N_CONV, HIDDEN, 1), lambda w, b, c: (0, 0, 0)),
            pl.BlockSpec((N_CONV, HIDDEN, HIDDEN), lambda w, b, c: (0, 0, 0)),
            pl.BlockSpec((N_CONV, HIDDEN, 1), lambda w, b, c: (0, 0, 0)),
        ],
        out_specs=[
            pl.BlockSpec((1, HIDDEN, BS),
                         lambda w, b, c: (w, 0, clamp(w, b, c)))
            for _ in range(N_CONV)
        ],
    )
    return pl.pallas_call(
        _filters_kernel,
        grid_spec=grid_spec,
        out_shape=[jax.ShapeDtypeStruct((NW, HIDDEN, CAP), _f32)
                   for _ in range(N_CONV)],
    )(cnts, d2.reshape(NW * (CAP // BS), 1, BS), Wf1T, bf1c, Wf2T, bf2c)


def _h_kernel(first, z_ref, emb_ref, hprev_ref, msg_ref, Wc2_ref, bc2_ref,
              Wc1_ref, bc1_ref, h_ref, hj_ref):
    if first:
        zcol = z_ref[...]
        oh = (zcol == lax.broadcasted_iota(_i32, (N_ATOMS, 10), 1)).astype(_f32)
        h = lax.dot_general(oh, emb_ref[...], (((1,), (0,)), ((), ())),
                            preferred_element_type=_f32, precision=lax.Precision.HIGHEST)
    else:
        m = lax.dot_general(msg_ref[...], Wc2_ref[...], (((1,), (0,)), ((), ())),
                            preferred_element_type=_f32, precision=lax.Precision.HIGHEST)
        h = hprev_ref[...] + _ssp(m + bc2_ref[...])
    h_ref[...] = h
    hj = lax.dot_general(h, Wc1_ref[...], (((1,), (0,)), ((), ())),
                         preferred_element_type=_f32, precision=lax.Precision.HIGHEST) + bc1_ref[...]
    hj_ref[...] = jnp.concatenate(
        [hj, jnp.zeros((N_ATOMS, 128 - HIDDEN), _f32)], axis=1)


def _h_call(first, z2, emb, hprev, msg, Wc2, bc2, Wc1, bc1r):
    return pl.pallas_call(
        functools.partial(_h_kernel, first),
        out_shape=[
            jax.ShapeDtypeStruct((N_ATOMS, HIDDEN), _f32),
            jax.ShapeDtypeStruct((N_ATOMS, 128), _f32),
        ],
    )(z2, emb, hprev, msg, Wc2, bc2, Wc1, bc1r)


def _msg_call(src, dst, cnts, f, hj):
    mesh = plsc.VectorSubcoreMesh(core_axis_name="c", subcore_axis_name="s")

    @functools.partial(
        pl.kernel,
        mesh=mesh,
        compiler_params=pltpu.CompilerParams(needs_layout_passes=False),
        out_type=jax.ShapeDtypeStruct((N_ATOMS * HIDDEN,), _f32),
        scratch_types=[
            pltpu.VMEM((4 * EC,), _i32),
            pltpu.VMEM((4 * EC,), _i32),
            pltpu.VMEM((4 * HIDDEN, EC), _f32),
            pltpu.VMEM((4 * EC, 128), _f32),
            pltpu.VMEM((ACC_ROWS * HIDDEN,), _f32),
            pltpu.VMEM((16,), _i32),
            pltpu.SemaphoreType.DMA,
            pltpu.SemaphoreType.DMA,
            pltpu.SemaphoreType.DMA,
            pltpu.SemaphoreType.DMA,
            pltpu.SemaphoreType.DMA,
            pltpu.SemaphoreType.DMA,
            pltpu.SemaphoreType.DMA,
            pltpu.SemaphoreType.DMA,
        ],
    )
    def msg_k(src_hbm, dst_hbm, cnt_hbm, f_hbm, hj_hbm, msg_hbm,
              idx_v, dst_v, f_v, rows_v, acc_v, cnt_v,
              semL0, semL1, semL2, semL3, semG0, semG1, semG2, semG3):
        wid = lax.axis_index("s") * 2 + lax.axis_index("c")
        semL = [semL0, semL1, semL2, semL3]
        semG = [semG0, semG1, semG2, semG3]

        def zero_body(k, _):
            acc_v[pl.ds(k * 16, 16)] = jnp.zeros((16,), _f32)
            return 0

        lax.fori_loop(0, ACC_ROWS * HIDDEN // 16, zero_body, 0, unroll=4)

        pltpu.sync_copy(cnt_hbm.at[pl.ds(wid * 16, 16)], cnt_v)
        cnt = jnp.max(cnt_v[...])
        nchunks = (cnt + (EC - 1)) // EC
        lane = lax.iota(_i32, 16)

        def L_descr(c, sl):
            e0 = c * EC
            return (
                pltpu.make_async_copy(src_hbm.at[pl.ds(wid * CAP + e0, EC)],
                                      idx_v.at[pl.ds(sl * EC, EC)], semL[sl]),
                pltpu.make_async_copy(dst_hbm.at[pl.ds(wid * CAP + e0, EC)],
                                      dst_v.at[pl.ds(sl * EC, EC)], semL[sl]),
                pltpu.make_async_copy(f_hbm.at[wid, :, pl.ds(e0, EC)],
                                      f_v.at[pl.ds(sl * HIDDEN, HIDDEN), :], semL[sl]),
            )

        def startL(c, sl):
            for d in L_descr(c, sl):
                d.start()

        def waitL(c, sl):
            for d in L_descr(c, sl):
                d.wait()

        def G_descr(sl):
            return pltpu.make_async_copy(
                hj_hbm.at[idx_v.at[pl.ds(sl * EC, EC)]],
                rows_v.at[pl.ds(sl * EC, EC), :], semG[sl])

        def compute(c, sl):
            rq = [_full(sl * HIDDEN + q * 16) + lane for q in range(4)]
            aq = [_full(q * 16) + lane for q in range(4)]

            def group_body(g, _):
                for k in range(16):
                    e = g * 16 + k
                    ev = _full(sl * EC) + e
                    dloc = plsc.load_gather(dst_v, [ev])
                    abase = dloc * HIDDEN
                    ecol = _full(0) + e
                    for q in range(4):
                        fq = plsc.load_gather(f_v, [rq[q], ecol])
                        hq = rows_v[sl * EC + e, pl.ds(q * 16, 16)]
                        plsc.addupdate_scatter(acc_v, [abase + aq[q]], fq * hq)
                return 0

            lax.fori_loop(0, EC // 16, group_body, 0)

        for w in range(4):
            @pl.when(nchunks > w)
            def _(w=w):
                startL(w, w)

        for w in range(2):
            @pl.when(nchunks > w)
            def _(w=w):
                waitL(w, w)
                G_descr(w).start()

        def super_body(qi, _):
            c0 = qi * 4
            for j in range(4):
                c = c0 + j

                @pl.when(c + 2 < nchunks)
                def _():
                    waitL(c + 2, (j + 2) % 4)
                    G_descr((j + 2) % 4).start()

                @pl.when(c < nchunks)
                def _():
                    G_descr(j).wait()
                    compute(c, j)

                @pl.when(c + 4 < nchunks)
                def _():
                    startL(c + 4, j)
            return 0

        lax.fori_loop(0, (nchunks + 3) // 4, super_body, 0)
        pltpu.sync_copy(acc_v.at[pl.ds(0, APW * HIDDEN)],
                        msg_hbm.at[pl.ds(wid * APW * HIDDEN, APW * HIDDEN)])

    return msg_k(src, dst, cnts, f, hj)


def _final_kernel(h_ref, msg_ref, Wc2_ref, bc2_ref, Wo1_ref, bo1_ref,
                  Wo2_ref, bo2_ref, out_ref):
    m = lax.dot_general(msg_ref[...], Wc2_ref[...], (((1,), (0,)), ((), ())),
                        preferred_element_type=_f32, precision=lax.Precision.HIGHEST)
    h = h_ref[...] + _ssp(m + bc2_ref[...])
    a1 = _ssp(lax.dot_general(h, Wo1_ref[...], (((1,), (0,)), ((), ())),
                              preferred_element_type=_f32, precision=lax.Precision.HIGHEST) + bo1_ref[...])
    e = lax.dot_general(a1, Wo2_ref[...], (((1,), (0,)), ((), ())),
                        preferred_element_type=_f32, precision=lax.Precision.HIGHEST) + bo2_ref[...]
    out_ref[...] = jnp.sum(e).reshape(1, 1)


def _final_call(h, msg, Wc2, bc2, Wo1, bo1r, Wo2, bo2r):
    return pl.pallas_call(
        _final_kernel,
        out_shape=jax.ShapeDtypeStruct((1, 1), _f32),
    )(h, msg, Wc2, bc2, Wo1, bo1r, Wo2, bo2r)


def kernel(xyz, emb, Wf1, bf1, Wf2, bf2, Wc1, bc1, Wc2, bc2, Wo1, bo1, Wo2, bo2, z):
    xyzf = xyz.astype(_f32)
    src_a, dst_a, d2, cnts = _nbr_call(xyzf[:, 0], xyzf[:, 1], xyzf[:, 2])

    z2 = z.astype(_i32).reshape(N_ATOMS, 1)
    dummy_h = jnp.zeros((N_ATOMS, HIDDEN), _f32)
    dummy_w = jnp.zeros((HIDDEN, HIDDEN), _f32)
    dummy_b = jnp.zeros((1, HIDDEN), _f32)

    Wf1T = jnp.transpose(Wf1, (0, 2, 1))
    bf1c = bf1.reshape(N_CONV, HIDDEN, 1)
    Wf2T = jnp.transpose(Wf2, (0, 2, 1))
    bf2c = bf2.reshape(N_CONV, HIDDEN, 1)

    fs = _filters_call(cnts, d2, Wf1T, bf1c, Wf2T, bf2c)

    h = dummy_h
    msg = dummy_h
    for l in range(N_CONV):
        first = l == 0
        h, hj = _h_call(
            first, z2, emb, h, msg,
            dummy_w if first else Wc2[l - 1],
            dummy_b if first else bc2[l - 1].reshape(1, HIDDEN),
            Wc1[l], bc1[l].reshape(1, HIDDEN),
        )
        msg = _msg_call(src_a, dst_a, cnts, fs[l], hj).reshape(N_ATOMS, HIDDEN)

    out = _final_call(h, msg, Wc2[N_CONV - 1], bc2[N_CONV - 1].reshape(1, HIDDEN),
                      Wo1, bo1.reshape(1, HIDDEN // 2), Wo2, bo2.reshape(1, 1))
    return out[0, 0]
